# baseline (speedup 1.0000x reference)
import jax
import jax.numpy as jnp
from jax import lax
from jax.experimental import pallas as pl
from jax.experimental.pallas import tpu as pltpu
from jax.experimental.pallas import tpu_sc as plsc

H = 512
W = 512
P = H * W
NC = 2
NS = 16
NW = NC * NS
PPW = P // NW
NB = 2
BATCH = PPW // NB
NQ = 4
BQ = BATCH // NQ
CHQ = BQ // 16
MAGIC = 8388608.0


def _sc_euler(tabx_hbm, taby_hbm, nv_hbm, out_hbm, nv, idx0, idx1, idx2, idx3,
              gx, gy, dcx, dcy, ax, ay, tabsx, tabsy, sem0, sem1, sem2, sem3):
    idxq = (idx0, idx1, idx2, idx3)
    semq = (sem0, sem1, sem2, sem3)
    c = lax.axis_index("c")
    s = lax.axis_index("s")
    wid = c * NS + s
    base = wid * PPW

    seg = P // NS
    pltpu.async_copy(tabx_hbm.at[pl.ds(s * seg, seg)], tabsx.at[pl.ds(s * seg, seg)], sem0).wait()
    pltpu.async_copy(taby_hbm.at[pl.ds(s * seg, seg)], tabsy.at[pl.ds(s * seg, seg)], sem1).wait()
    plsc.subcore_barrier()

    pltpu.sync_copy(nv_hbm, nv)
    n = nv[...][0]
    iota = lax.iota(jnp.int32, 16)

    def wait_q(q):
        off = q * BQ
        pltpu.make_async_copy(tabx_hbm.at[pl.ds(0, BQ)], gx.at[pl.ds(off, BQ)], semq[q]).wait()
        pltpu.make_async_copy(taby_hbm.at[pl.ds(0, BQ)], gy.at[pl.ds(off, BQ)], semq[q]).wait()

    def fire_q(q):
        off = q * BQ
        pltpu.async_copy(tabsx.at[idxq[q].at[0]], gx.at[pl.ds(off, BQ)], semq[q])
        pltpu.async_copy(tabsy.at[idxq[q].at[0]], gy.at[pl.ds(off, BQ)], semq[q])

    for b in range(NB):
        bbase = base + b * BATCH

        def make_pass(off, idxr, first):
            def compute_chunk(j):
                sl = pl.ds(off + j * 16, 16)
                p = bbase + off + j * 16 + iota
                cx = (p & (W - 1)).astype(jnp.float32)
                cy = (p >> 9).astype(jnp.float32)
                gxv = gx[sl]
                gyv = gy[sl]
                if first:
                    ax[sl] = jnp.zeros((16,), jnp.float32)
                    ay[sl] = jnp.zeros((16,), jnp.float32)
                    tx = cx + gxv
                    ty = cy + gyv
                    mb0 = None
                else:
                    ax[sl] = ax[sl] + gxv
                    ay[sl] = ay[sl] + gyv
                    dxl = dcx[sl]
                    dyl = dcy[sl]
                    mb0 = dxl < -0.5
                    tx = jnp.where(mb0, cx, dxl) + gxv
                    ty = dyl + gyv
                oob = (tx > W - 1.0) | (tx < 0.0) | (ty > H - 1.0) | (ty < 0.0)
                m = oob if first else (mb0 | oob)
                dxe = jnp.where(m, cx, tx)
                dye = jnp.where(m, cy, ty)
                dcx[sl] = jnp.where(m, -1.0 - cx, tx)
                dcy[sl] = dye
                rx = ((dxe + MAGIC) - MAGIC).astype(jnp.int32)
                ry = ((dye + MAGIC) - MAGIC).astype(jnp.int32)
                idxr[0, pl.ds(j * 16, 16)] = (ry << 9) | rx
            return compute_chunk

        passes0 = [make_pass(q * BQ, idxq[q], True) for q in range(NQ)]
        passes = [make_pass(q * BQ, idxq[q], False) for q in range(NQ)]

        for q in range(NQ):
            pltpu.async_copy(tabsx.at[pl.ds(bbase + q * BQ, BQ)], gx.at[pl.ds(q * BQ, BQ)], semq[q])
            pltpu.async_copy(tabsy.at[pl.ds(bbase + q * BQ, BQ)], gy.at[pl.ds(q * BQ, BQ)], semq[q])

        for q in range(NQ):
            wait_q(q)
            plsc.parallel_loop(0, CHQ, unroll=4)(passes0[q])
            fire_q(q)

        def iter_body(it, _):
            for q in range(NQ):
                wait_q(q)
                plsc.parallel_loop(0, CHQ, unroll=4)(passes[q])
                fire_q(q)
            return 0

        lax.fori_loop(1, n, iter_body, 0)

        for q in range(NQ):
            wait_q(q)

        @plsc.parallel_loop(0, BATCH // 16, unroll=4)
        def fin_chunk(j):
            sl = pl.ds(j * 16, 16)
            ax[sl] = ax[sl] + gx[sl]
            ay[sl] = ay[sl] + gy[sl]

        pltpu.sync_copy(ax, out_hbm.at[0, pl.ds(bbase, BATCH)])
        pltpu.sync_copy(ay, out_hbm.at[1, pl.ds(bbase, BATCH)])


@jax.jit
def kernel(motion, destination_frame):
    tabx = motion[0, 0].reshape(P).astype(jnp.float32)
    taby = motion[0, 1].reshape(P).astype(jnp.float32)
    nvec = jnp.broadcast_to(destination_frame.astype(jnp.int32).reshape(1), (16,))
    mesh = plsc.VectorSubcoreMesh(core_axis_name="c", subcore_axis_name="s")
    out = pl.kernel(
        _sc_euler,
        out_type=jax.ShapeDtypeStruct((2, P), jnp.float32),
        mesh=mesh,
        scratch_types=[
            pltpu.VMEM((16,), jnp.int32),
            pltpu.VMEM((1, BQ), jnp.int32),
            pltpu.VMEM((1, BQ), jnp.int32),
            pltpu.VMEM((1, BQ), jnp.int32),
            pltpu.VMEM((1, BQ), jnp.int32),
            pltpu.VMEM((BATCH,), jnp.float32),
            pltpu.VMEM((BATCH,), jnp.float32),
            pltpu.VMEM((BATCH,), jnp.float32),
            pltpu.VMEM((BATCH,), jnp.float32),
            pltpu.VMEM((BATCH,), jnp.float32),
            pltpu.VMEM((BATCH,), jnp.float32),
            pltpu.VMEM_SHARED((P,), jnp.float32),
            pltpu.VMEM_SHARED((P,), jnp.float32),
            pltpu.SemaphoreType.DMA,
            pltpu.SemaphoreType.DMA,
            pltpu.SemaphoreType.DMA,
            pltpu.SemaphoreType.DMA,
        ],
    )(tabx, taby, nvec)
    return out.reshape(1, 2, H, W)


# speedup vs baseline: 1.0461x; 1.0461x over previous
import jax
import jax.numpy as jnp
from jax import lax
from jax.experimental import pallas as pl
from jax.experimental.pallas import tpu as pltpu
from jax.experimental.pallas import tpu_sc as plsc

H = 512
W = 512
P = H * W
NC = 2
NS = 16
NW = NC * NS
PPW = P // NW
NB = 2
BATCH = PPW // NB
BHALF = BATCH // 2
CHH = BHALF // 16
MAGIC = 8388608.0


def _sc_euler(tabx_hbm, taby_hbm, nv_hbm, out_hbm, nv, idxa, idxb, g,
              dcx, dcy, ax, ay, tabs, sema, semb):
    c = lax.axis_index("c")
    s = lax.axis_index("s")
    wid = c * NS + s
    base = wid * PPW

    seg = P // NS
    pltpu.async_copy(tabx_hbm.at[pl.ds(s * seg, seg)], tabs.at[pl.ds(s * seg, seg)], sema).wait()
    pltpu.async_copy(taby_hbm.at[pl.ds(s * seg, seg)], tabs.at[pl.ds(P + s * seg, seg)], semb).wait()
    plsc.subcore_barrier()

    pltpu.sync_copy(nv_hbm, nv)
    n = nv[...][0]
    iota = lax.iota(jnp.int32, 16)

    def wait_half(sem, h):
        goff = h * 2 * BHALF
        pltpu.make_async_copy(tabx_hbm.at[pl.ds(0, 2 * BHALF)],
                              g.at[pl.ds(goff, 2 * BHALF)], sem).wait()

    def fire_half(sem, h, idxr):
        goff = h * 2 * BHALF
        pltpu.async_copy(tabs.at[idxr.at[0]], g.at[pl.ds(goff, 2 * BHALF)], sem)

    for b in range(NB):
        bbase = base + b * BATCH

        def make_pass(h, idxr, first):
            goff = h * 2 * BHALF
            poff = h * BHALF

            def compute_chunk(j):
                sl = pl.ds(poff + j * 16, 16)
                p = bbase + poff + j * 16 + iota
                cx = (p & (W - 1)).astype(jnp.float32)
                cy = (p >> 9).astype(jnp.float32)
                gxv = g[pl.ds(goff + j * 16, 16)]
                gyv = g[pl.ds(goff + BHALF + j * 16, 16)]
                if first:
                    ax[sl] = jnp.zeros((16,), jnp.float32)
                    ay[sl] = jnp.zeros((16,), jnp.float32)
                    tx = cx + gxv
                    ty = cy + gyv
                    mb0 = None
                else:
                    ax[sl] = ax[sl] + gxv
                    ay[sl] = ay[sl] + gyv
                    dxl = dcx[sl]
                    dyl = dcy[sl]
                    mb0 = dxl < -0.5
                    tx = jnp.where(mb0, cx, dxl) + gxv
                    ty = dyl + gyv
                oob = (tx > W - 1.0) | (tx < 0.0) | (ty > H - 1.0) | (ty < 0.0)
                m = oob if first else (mb0 | oob)
                dxe = jnp.where(m, cx, tx)
                dye = jnp.where(m, cy, ty)
                dcx[sl] = jnp.where(m, -1.0 - cx, tx)
                dcy[sl] = dye
                rx = ((dxe + MAGIC) - MAGIC).astype(jnp.int32)
                ry = ((dye + MAGIC) - MAGIC).astype(jnp.int32)
                idx = (ry << 9) | rx
                idxr[0, pl.ds(j * 16, 16)] = idx
                idxr[0, pl.ds(BHALF + j * 16, 16)] = idx + P
            return compute_chunk

        pass_a0 = make_pass(0, idxa, True)
        pass_b0 = make_pass(1, idxb, True)
        pass_a = make_pass(0, idxa, False)
        pass_b = make_pass(1, idxb, False)

        pltpu.async_copy(tabs.at[pl.ds(bbase, BHALF)], g.at[pl.ds(0, BHALF)], sema)
        pltpu.async_copy(tabs.at[pl.ds(P + bbase, BHALF)], g.at[pl.ds(BHALF, BHALF)], sema)
        pltpu.async_copy(tabs.at[pl.ds(bbase + BHALF, BHALF)], g.at[pl.ds(2 * BHALF, BHALF)], semb)
        pltpu.async_copy(tabs.at[pl.ds(P + bbase + BHALF, BHALF)], g.at[pl.ds(3 * BHALF, BHALF)], semb)

        wait_half(sema, 0)
        plsc.parallel_loop(0, CHH, unroll=4)(pass_a0)
        fire_half(sema, 0, idxa)
        wait_half(semb, 1)
        plsc.parallel_loop(0, CHH, unroll=4)(pass_b0)
        fire_half(semb, 1, idxb)

        def iter_body(it, _):
            wait_half(sema, 0)
            plsc.parallel_loop(0, CHH, unroll=4)(pass_a)
            fire_half(sema, 0, idxa)
            wait_half(semb, 1)
            plsc.parallel_loop(0, CHH, unroll=4)(pass_b)
            fire_half(semb, 1, idxb)
            return 0

        lax.fori_loop(1, n, iter_body, 0)

        wait_half(sema, 0)
        wait_half(semb, 1)

        for h in range(2):
            goff = h * 2 * BHALF
            poff = h * BHALF

            @plsc.parallel_loop(0, CHH, unroll=4)
            def fin_chunk(j):
                sl = pl.ds(poff + j * 16, 16)
                ax[sl] = ax[sl] + g[pl.ds(goff + j * 16, 16)]
                ay[sl] = ay[sl] + g[pl.ds(goff + BHALF + j * 16, 16)]

        pltpu.sync_copy(ax, out_hbm.at[0, pl.ds(bbase, BATCH)])
        pltpu.sync_copy(ay, out_hbm.at[1, pl.ds(bbase, BATCH)])


@jax.jit
def kernel(motion, destination_frame):
    tabx = motion[0, 0].reshape(P).astype(jnp.float32)
    taby = motion[0, 1].reshape(P).astype(jnp.float32)
    nvec = jnp.broadcast_to(destination_frame.astype(jnp.int32).reshape(1), (16,))
    mesh = plsc.VectorSubcoreMesh(core_axis_name="c", subcore_axis_name="s")
    out = pl.kernel(
        _sc_euler,
        out_type=jax.ShapeDtypeStruct((2, P), jnp.float32),
        mesh=mesh,
        scratch_types=[
            pltpu.VMEM((16,), jnp.int32),
            pltpu.VMEM((1, 2 * BHALF), jnp.int32),
            pltpu.VMEM((1, 2 * BHALF), jnp.int32),
            pltpu.VMEM((2 * BATCH,), jnp.float32),
            pltpu.VMEM((BATCH,), jnp.float32),
            pltpu.VMEM((BATCH,), jnp.float32),
            pltpu.VMEM((BATCH,), jnp.float32),
            pltpu.VMEM((BATCH,), jnp.float32),
            pltpu.VMEM_SHARED((2 * P,), jnp.float32),
            pltpu.SemaphoreType.DMA,
            pltpu.SemaphoreType.DMA,
        ],
    )(tabx, taby, nvec)
    return out.reshape(1, 2, H, W)
